# baseline (speedup 1.0000x reference)
import functools
import math

import jax
import jax.numpy as jnp
from jax import lax
from jax.experimental import pallas as pl
from jax.experimental.pallas import tpu as pltpu
from jax.experimental.pallas import tpu_sc as plsc

DIM = 128
LANES = 16
SCALE = math.sqrt(float(DIM))
NUM_CORES = 2
NUM_SUBCORES = 16
NW = NUM_CORES * NUM_SUBCORES
CHUNK = 64
NBUF = 3


def _emb_body(seq_len, rows_per_w, ctx_hbm, we_hbm, pos_hbm, out_hbm,
              idx_v, rows_buf, pos_buf, pos_sh, isem, ssem,
              gsem0, gsem1, gsem2, psem0, psem1, psem2,
              osem0, osem1, osem2):
    n_chunks = rows_per_w // CHUNK
    w_per_row = seq_len // rows_per_w
    w_half = w_per_row // NUM_CORES
    gsems = (gsem0, gsem1, gsem2)
    psems = (psem0, psem1, psem2)
    osems = (osem0, osem1, osem2)
    c = lax.axis_index("c")
    s = lax.axis_index("s")
    wid = s * NUM_CORES + c
    bb = wid // w_per_row
    col = lax.rem(wid, w_per_row) * rows_per_w

    idx_cp = pltpu.async_copy(ctx_hbm.at[bb, pl.ds(col, rows_per_w)],
                              idx_v, isem)

    q = s // w_half
    r = lax.rem(s, w_half)
    gstart = q * (NUM_CORES * rows_per_w) + c * rows_per_w + r * CHUNK
    stage_cp = pltpu.async_copy(pos_hbm.at[pl.ds(gstart, CHUNK)],
                                pos_sh.at[pl.ds(s * CHUNK, CHUNK)], ssem)
    idx_cp.wait()

    pos_local = r * rows_per_w

    def fire_g(k):
        return pltpu.async_copy(we_hbm.at[idx_v.at[pl.ds(k * CHUNK, CHUNK)]],
                                rows_buf.at[k % NBUF], gsems[k % NBUF])

    def fire_p(k):
        return pltpu.async_copy(
            pos_sh.at[pl.ds(pos_local + k * CHUNK, CHUNK)],
            pos_buf.at[k % NBUF], psems[k % NBUF])

    def compute(k, lo, hi):
        rb = rows_buf.at[k % NBUF]
        pb = pos_buf.at[k % NBUF]

        def row_body(rr, carry):
            for jj in range(2):
                for j in range(DIM // LANES):
                    sl = pl.ds(j * LANES, LANES)
                    rb[2 * rr + jj, sl] = (rb[2 * rr + jj, sl] * SCALE
                                           + pb[2 * rr + jj, sl])
            return carry

        lax.fori_loop(lo // 2, hi // 2, row_body, 0)

    def fma(k):
        compute(k, 0, CHUNK)
        return pltpu.async_copy(rows_buf.at[k % NBUF],
                                out_hbm.at[bb, pl.ds(col + k * CHUNK, CHUNK)],
                                osems[k % NBUF])

    g_cp = {k: fire_g(k) for k in range(min(NBUF, n_chunks))}
    stage_cp.wait()
    plsc.subcore_barrier()
    p_cp = {k: fire_p(k) for k in range(min(NBUF, n_chunks))}
    out_cp = {}
    last = n_chunks - 1
    for k in range(n_chunks):
        g_cp.pop(k).wait()
        p_cp.pop(k).wait()
        j = k + NBUF - 1
        if k >= 1 and j < n_chunks:
            out_cp.pop(k - 1).wait()
            g_cp[j] = fire_g(j)
            p_cp[j] = fire_p(j)
        if k < last:
            out_cp[k] = fma(k)
    half = CHUNK // 2
    compute(last, 0, half)
    tail0 = pltpu.async_copy(rows_buf.at[last % NBUF, pl.ds(0, half)],
                             out_hbm.at[bb, pl.ds(col + last * CHUNK, half)],
                             osems[last % NBUF])
    compute(last, half, CHUNK)
    tail1 = pltpu.async_copy(
        rows_buf.at[last % NBUF, pl.ds(half, half)],
        out_hbm.at[bb, pl.ds(col + last * CHUNK + half, half)], isem)
    for k in sorted(out_cp):
        out_cp.pop(k).wait()
    tail0.wait()
    tail1.wait()


def kernel(context, word_emb, pos_emb):
    b, l = context.shape
    rows_per_w = (b * l) // NW
    ctx = context.astype(jnp.int32)
    n_chunks = rows_per_w // CHUNK

    mesh = plsc.VectorSubcoreMesh(core_axis_name="c", subcore_axis_name="s")
    body = functools.partial(_emb_body, l, rows_per_w)
    return pl.kernel(
        body,
        mesh=mesh,
        out_type=jax.ShapeDtypeStruct((b, l, DIM), jnp.float32),
        scratch_types=[
            pltpu.VMEM((n_chunks * CHUNK,), jnp.int32),
            pltpu.VMEM((NBUF, CHUNK, DIM), jnp.float32),
            pltpu.VMEM((NBUF, CHUNK, DIM), jnp.float32),
            pltpu.VMEM_SHARED((NUM_SUBCORES * CHUNK, DIM), jnp.float32),
            pltpu.SemaphoreType.DMA,
            pltpu.SemaphoreType.DMA,
            pltpu.SemaphoreType.DMA,
            pltpu.SemaphoreType.DMA,
            pltpu.SemaphoreType.DMA,
            pltpu.SemaphoreType.DMA,
            pltpu.SemaphoreType.DMA,
            pltpu.SemaphoreType.DMA,
            pltpu.SemaphoreType.DMA,
            pltpu.SemaphoreType.DMA,
            pltpu.SemaphoreType.DMA,
        ],
    )(ctx, word_emb, pos_emb)


# speedup vs baseline: 1.0134x; 1.0134x over previous
import functools
import math

import jax
import jax.numpy as jnp
from jax import lax
from jax.experimental import pallas as pl
from jax.experimental.pallas import tpu as pltpu
from jax.experimental.pallas import tpu_sc as plsc

DIM = 128
LANES = 16
SCALE = math.sqrt(float(DIM))
NUM_CORES = 2
NUM_SUBCORES = 16
NW = NUM_CORES * NUM_SUBCORES
CHUNK = 64
NBUF = 3


def _emb_body(seq_len, rows_per_w, ctx_hbm, we_hbm, pos_hbm, out_hbm,
              idx_v, rows_buf, pos_buf, pos_sh, isem, ssem,
              gsem0, gsem1, gsem2, psem0, psem1, psem2,
              osem0, osem1, osem2):
    n_chunks = rows_per_w // CHUNK
    w_per_row = seq_len // rows_per_w
    w_half = w_per_row // NUM_CORES
    gsems = (gsem0, gsem1, gsem2)
    psems = (psem0, psem1, psem2)
    osems = (osem0, osem1, osem2)
    c = lax.axis_index("c")
    s = lax.axis_index("s")
    wid = s * NUM_CORES + c
    bb = wid // w_per_row
    col = lax.rem(wid, w_per_row) * rows_per_w

    idx_cp = pltpu.async_copy(ctx_hbm.at[bb, pl.ds(col, rows_per_w)],
                              idx_v, isem)

    q = s // w_half
    r = lax.rem(s, w_half)
    gstart = q * (NUM_CORES * rows_per_w) + c * rows_per_w + r * CHUNK
    stage_cp = pltpu.async_copy(pos_hbm.at[pl.ds(gstart, CHUNK)],
                                pos_sh.at[pl.ds(s * CHUNK, CHUNK)], ssem)
    idx_cp.wait()

    pos_local = r * rows_per_w

    def fire_g(k):
        return pltpu.async_copy(we_hbm.at[idx_v.at[pl.ds(k * CHUNK, CHUNK)]],
                                rows_buf.at[k % NBUF], gsems[k % NBUF])

    def fire_p(k):
        return pltpu.async_copy(
            pos_sh.at[pl.ds(pos_local + k * CHUNK, CHUNK)],
            pos_buf.at[k % NBUF], psems[k % NBUF])

    def compute(k, lo, hi):
        rb = rows_buf.at[k % NBUF]
        pb = pos_buf.at[k % NBUF]

        def row_body(rr, carry):
            for jj in range(2):
                for j in range(DIM // LANES):
                    sl = pl.ds(j * LANES, LANES)
                    rb[2 * rr + jj, sl] = (rb[2 * rr + jj, sl] * SCALE
                                           + pb[2 * rr + jj, sl])
            return carry

        lax.fori_loop(lo // 2, hi // 2, row_body, 0)

    def fma(k):
        compute(k, 0, CHUNK)
        return pltpu.async_copy(rows_buf.at[k % NBUF],
                                out_hbm.at[bb, pl.ds(col + k * CHUNK, CHUNK)],
                                osems[k % NBUF])

    g_cp = {0: fire_g(0), 1: fire_g(1)}
    stage_cp.wait()
    plsc.subcore_barrier()
    p_cp = {0: fire_p(0), 1: fire_p(1)}
    out_cp = {}
    last = n_chunks - 1
    for k in range(n_chunks):
        g_cp.pop(k).wait()
        p_cp.pop(k).wait()
        if k + 2 < n_chunks:
            if k >= 1:
                out_cp.pop(k - 1).wait()
            g_cp[k + 2] = fire_g(k + 2)
            p_cp[k + 2] = fire_p(k + 2)
        if k < last:
            out_cp[k] = fma(k)
    half = CHUNK // 2
    compute(last, 0, half)
    tail0 = pltpu.async_copy(rows_buf.at[last % NBUF, pl.ds(0, half)],
                             out_hbm.at[bb, pl.ds(col + last * CHUNK, half)],
                             osems[last % NBUF])
    compute(last, half, CHUNK)
    tail1 = pltpu.async_copy(
        rows_buf.at[last % NBUF, pl.ds(half, half)],
        out_hbm.at[bb, pl.ds(col + last * CHUNK + half, half)], isem)
    for k in sorted(out_cp):
        out_cp.pop(k).wait()
    tail0.wait()
    tail1.wait()


def kernel(context, word_emb, pos_emb):
    b, l = context.shape
    rows_per_w = (b * l) // NW
    ctx = context.astype(jnp.int32)
    n_chunks = rows_per_w // CHUNK

    mesh = plsc.VectorSubcoreMesh(core_axis_name="c", subcore_axis_name="s")
    body = functools.partial(_emb_body, l, rows_per_w)
    return pl.kernel(
        body,
        mesh=mesh,
        out_type=jax.ShapeDtypeStruct((b, l, DIM), jnp.float32),
        scratch_types=[
            pltpu.VMEM((n_chunks * CHUNK,), jnp.int32),
            pltpu.VMEM((NBUF, CHUNK, DIM), jnp.float32),
            pltpu.VMEM((NBUF, CHUNK, DIM), jnp.float32),
            pltpu.VMEM_SHARED((NUM_SUBCORES * CHUNK, DIM), jnp.float32),
            pltpu.SemaphoreType.DMA,
            pltpu.SemaphoreType.DMA,
            pltpu.SemaphoreType.DMA,
            pltpu.SemaphoreType.DMA,
            pltpu.SemaphoreType.DMA,
            pltpu.SemaphoreType.DMA,
            pltpu.SemaphoreType.DMA,
            pltpu.SemaphoreType.DMA,
            pltpu.SemaphoreType.DMA,
            pltpu.SemaphoreType.DMA,
            pltpu.SemaphoreType.DMA,
        ],
    )(ctx, word_emb, pos_emb)
